# baseline (speedup 1.0000x reference)
import functools

import jax
import jax.numpy as jnp
from jax.experimental import pallas as pl
from jax.experimental.pallas import tpu as pltpu


B, S, KD, VD, CD, BUF = 4, 4096, 128, 128, 32, 4096

_DIMNUM_C1C1 = (((1,), (1,)), ((), ()))

_NC = 8
_C = (B * S) // _NC


def _kv_kernel(k_hbm, v_hbm, m_ref, wk_ref, wkr_ref, wv_ref, wvr_ref,
               ok_hbm, ov_hbm, om_ref, kbuf, vbuf, sems):
    om_ref[...] = m_ref[...] != 0.0
    for i in range(_NC):
        rows = pl.ds(i * _C, _C)
        pltpu.make_async_copy(k_hbm.at[rows, :], kbuf.at[i], sems.at[0, i]).start()
        pltpu.make_async_copy(v_hbm.at[rows, :], vbuf.at[i], sems.at[1, i]).start()
    for i in range(_NC):
        rows = pl.ds(i * _C, _C)
        pltpu.make_async_copy(k_hbm.at[rows, :], kbuf.at[i], sems.at[0, i]).wait()
        ck = jax.lax.dot_general(kbuf[i], wk_ref[...], _DIMNUM_C1C1,
                                 preferred_element_type=jnp.float32)
        kbuf[i] = jax.lax.dot_general(ck, wkr_ref[...], _DIMNUM_C1C1,
                                      preferred_element_type=jnp.float32)
        pltpu.make_async_copy(kbuf.at[i], ok_hbm.at[rows, :], sems.at[2, i]).start()
        pltpu.make_async_copy(v_hbm.at[rows, :], vbuf.at[i], sems.at[1, i]).wait()
        cv = jax.lax.dot_general(vbuf[i], wv_ref[...], _DIMNUM_C1C1,
                                 preferred_element_type=jnp.float32)
        vbuf[i] = jax.lax.dot_general(cv, wvr_ref[...], _DIMNUM_C1C1,
                                      preferred_element_type=jnp.float32)
        pltpu.make_async_copy(vbuf.at[i], ov_hbm.at[rows, :], sems.at[3, i]).start()
    for i in range(_NC):
        rows = pl.ds(i * _C, _C)
        pltpu.make_async_copy(kbuf.at[i], ok_hbm.at[rows, :], sems.at[2, i]).wait()
        pltpu.make_async_copy(vbuf.at[i], ov_hbm.at[rows, :], sems.at[3, i]).wait()


@jax.jit
def _run(keys, values, mask, Wk, Wk_rec, Wv, Wv_rec):
    rows = B * S
    k2 = keys.reshape(rows, KD)
    v2 = values.reshape(rows, VD)
    m2 = mask.reshape(rows, 1)
    any_spec = pl.BlockSpec(memory_space=pltpu.MemorySpace.HBM)
    vmem_spec = pl.BlockSpec(memory_space=pltpu.MemorySpace.VMEM)
    ok, ov, om = pl.pallas_call(
        _kv_kernel,
        in_specs=[any_spec, any_spec, vmem_spec,
                  vmem_spec, vmem_spec, vmem_spec, vmem_spec],
        out_specs=[any_spec, any_spec, vmem_spec],
        out_shape=[
            jax.ShapeDtypeStruct((rows, KD), jnp.float32),
            jax.ShapeDtypeStruct((rows, VD), jnp.float32),
            jax.ShapeDtypeStruct((rows, 1), jnp.bool_),
        ],
        scratch_shapes=[
            pltpu.VMEM((_NC, _C, KD), jnp.float32),
            pltpu.VMEM((_NC, _C, VD), jnp.float32),
            pltpu.SemaphoreType.DMA((4, _NC)),
        ],
        compiler_params=pltpu.CompilerParams(
            vmem_limit_bytes=100 * 1024 * 1024),
    )(k2, v2, m2, Wk, Wk_rec, Wv, Wv_rec)
    return (ok.reshape(B, BUF, KD), ov.reshape(B, BUF, VD),
            om.reshape(B, BUF))


def kernel(keys, values, mask, Wk, Wk_rec, Wv, Wv_rec):
    return _run(keys, values, mask, Wk, Wk_rec, Wv, Wv_rec)


# speedup vs baseline: 1.0261x; 1.0261x over previous
import functools

import jax
import jax.numpy as jnp
from jax.experimental import pallas as pl
from jax.experimental.pallas import tpu as pltpu


B, S, KD, VD, CD, BUF = 4, 4096, 128, 128, 32, 4096

_DIMNUM_C1C1 = (((1,), (1,)), ((), ()))

_NC = 4
_C = (B * S) // _NC


def _kv_kernel(k_hbm, v_hbm, m_ref, wk_ref, wkr_ref, wv_ref, wvr_ref,
               ok_hbm, ov_hbm, om_ref, kbuf, vbuf, sems):
    om_ref[...] = m_ref[...] != 0.0
    for i in range(_NC):
        rows = pl.ds(i * _C, _C)
        pltpu.make_async_copy(k_hbm.at[rows, :], kbuf.at[i], sems.at[0, i]).start()
        pltpu.make_async_copy(v_hbm.at[rows, :], vbuf.at[i], sems.at[1, i]).start()
    for i in range(_NC):
        rows = pl.ds(i * _C, _C)
        pltpu.make_async_copy(k_hbm.at[rows, :], kbuf.at[i], sems.at[0, i]).wait()
        ck = jax.lax.dot_general(kbuf[i], wk_ref[...], _DIMNUM_C1C1,
                                 preferred_element_type=jnp.float32)
        kbuf[i] = jax.lax.dot_general(ck, wkr_ref[...], _DIMNUM_C1C1,
                                      preferred_element_type=jnp.float32)
        pltpu.make_async_copy(kbuf.at[i], ok_hbm.at[rows, :], sems.at[2, i]).start()
        pltpu.make_async_copy(v_hbm.at[rows, :], vbuf.at[i], sems.at[1, i]).wait()
        cv = jax.lax.dot_general(vbuf[i], wv_ref[...], _DIMNUM_C1C1,
                                 preferred_element_type=jnp.float32)
        vbuf[i] = jax.lax.dot_general(cv, wvr_ref[...], _DIMNUM_C1C1,
                                      preferred_element_type=jnp.float32)
        pltpu.make_async_copy(vbuf.at[i], ov_hbm.at[rows, :], sems.at[3, i]).start()
    for i in range(_NC):
        rows = pl.ds(i * _C, _C)
        pltpu.make_async_copy(kbuf.at[i], ok_hbm.at[rows, :], sems.at[2, i]).wait()
        pltpu.make_async_copy(vbuf.at[i], ov_hbm.at[rows, :], sems.at[3, i]).wait()


@jax.jit
def _run(keys, values, mask, Wk, Wk_rec, Wv, Wv_rec):
    rows = B * S
    k2 = keys.reshape(rows, KD)
    v2 = values.reshape(rows, VD)
    m2 = mask.reshape(rows, 1)
    any_spec = pl.BlockSpec(memory_space=pltpu.MemorySpace.HBM)
    vmem_spec = pl.BlockSpec(memory_space=pltpu.MemorySpace.VMEM)
    ok, ov, om = pl.pallas_call(
        _kv_kernel,
        in_specs=[any_spec, any_spec, vmem_spec,
                  vmem_spec, vmem_spec, vmem_spec, vmem_spec],
        out_specs=[any_spec, any_spec, vmem_spec],
        out_shape=[
            jax.ShapeDtypeStruct((rows, KD), jnp.float32),
            jax.ShapeDtypeStruct((rows, VD), jnp.float32),
            jax.ShapeDtypeStruct((rows, 1), jnp.bool_),
        ],
        scratch_shapes=[
            pltpu.VMEM((_NC, _C, KD), jnp.float32),
            pltpu.VMEM((_NC, _C, VD), jnp.float32),
            pltpu.SemaphoreType.DMA((4, _NC)),
        ],
        compiler_params=pltpu.CompilerParams(
            vmem_limit_bytes=100 * 1024 * 1024),
    )(k2, v2, m2, Wk, Wk_rec, Wv, Wv_rec)
    return (ok.reshape(B, BUF, KD), ov.reshape(B, BUF, VD),
            om.reshape(B, BUF))


def kernel(keys, values, mask, Wk, Wk_rec, Wv, Wv_rec):
    return _run(keys, values, mask, Wk, Wk_rec, Wv, Wv_rec)


# speedup vs baseline: 1.0434x; 1.0169x over previous
import functools

import jax
import jax.numpy as jnp
from jax.experimental import pallas as pl
from jax.experimental.pallas import tpu as pltpu


B, S, KD, VD, CD, BUF = 4, 4096, 128, 128, 32, 4096

_DIMNUM_C1C1 = (((1,), (1,)), ((), ()))


def _kv_kernel(k_ref, v_ref, m_ref, wk_ref, wkr_ref, wv_ref, wvr_ref,
               ok_ref, ov_ref, om_ref):
    ck = jax.lax.dot_general(k_ref[...], wk_ref[...], _DIMNUM_C1C1,
                             preferred_element_type=jnp.float32)
    ok_ref[...] = jax.lax.dot_general(ck, wkr_ref[...], _DIMNUM_C1C1,
                                      preferred_element_type=jnp.float32)
    cv = jax.lax.dot_general(v_ref[...], wv_ref[...], _DIMNUM_C1C1,
                             preferred_element_type=jnp.float32)
    ov_ref[...] = jax.lax.dot_general(cv, wvr_ref[...], _DIMNUM_C1C1,
                                      preferred_element_type=jnp.float32)
    om_ref[...] = m_ref[...] != 0.0


@functools.partial(jax.jit, static_argnames=("blk",))
def _run(keys, values, mask, Wk, Wk_rec, Wv, Wv_rec, blk=8192):
    rows = B * S
    k2 = keys.reshape(rows, KD)
    v2 = values.reshape(rows, VD)
    m2 = mask.reshape(rows, 1)
    grid = (rows // blk,)
    row_spec = lambda d: pl.BlockSpec((blk, d), lambda i: (i, 0))
    full_spec = lambda a: pl.BlockSpec(a.shape, lambda i: (0, 0))
    ok, ov, om = pl.pallas_call(
        _kv_kernel,
        grid=grid,
        in_specs=[
            row_spec(KD),
            row_spec(VD),
            row_spec(1),
            full_spec(Wk),
            full_spec(Wk_rec),
            full_spec(Wv),
            full_spec(Wv_rec),
        ],
        out_specs=[row_spec(KD), row_spec(VD), row_spec(1)],
        out_shape=[
            jax.ShapeDtypeStruct((rows, KD), jnp.float32),
            jax.ShapeDtypeStruct((rows, VD), jnp.float32),
            jax.ShapeDtypeStruct((rows, 1), jnp.bool_),
        ],
        compiler_params=pltpu.CompilerParams(
            dimension_semantics=("parallel",),
            vmem_limit_bytes=100 * 1024 * 1024),
    )(k2, v2, m2, Wk, Wk_rec, Wv, Wv_rec)
    return (ok.reshape(B, BUF, KD), ov.reshape(B, BUF, VD),
            om.reshape(B, BUF))


def kernel(keys, values, mask, Wk, Wk_rec, Wv, Wv_rec):
    return _run(keys, values, mask, Wk, Wk_rec, Wv, Wv_rec)
